# baseline (speedup 1.0000x reference)
import functools

import jax
import jax.numpy as jnp
from jax import lax
from jax.experimental import pallas as pl
from jax.experimental.pallas import tpu as pltpu
from jax.experimental.pallas import tpu_sc as plsc

N = 100000
NC, NS, CB = 2, 16, 2048
HNP = 50432
TRASH = HNP
AGR = HNP + 128
RPT = AGR // NS
EPW = 204800
EP = EPW * NS
CHUNK = 2
RPW = EPW // CB
STEPS = RPW // CHUNK
BN = 2000


def _make_segsum(FW):
    mesh = plsc.VectorSubcoreMesh(
        core_axis_name="c", subcore_axis_name="s", num_cores=NC, num_subcores=NS
    )

    @functools.partial(
        pl.kernel,
        out_type=jax.ShapeDtypeStruct((NC, AGR, FW), jnp.float32),
        mesh=mesh,
        scratch_types=[
            pltpu.VMEM((CHUNK, CB), jnp.int32),
            pltpu.VMEM((CHUNK, CB), jnp.int32),
            pltpu.VMEM((CHUNK, CB, FW), jnp.float32),
            pltpu.VMEM_SHARED((AGR, FW), jnp.float32),
            pltpu.SemaphoreType.DMA,
            pltpu.SemaphoreType.DMA,
        ],
        compiler_params=pltpu.CompilerParams(use_tc_tiling_on_sc=False),
    )
    def segsum(h_hbm, src_hbm, dst_hbm, zero_hbm, out_hbm, sidx, didx, rows,
               agg, gsem, ssem):
        c = lax.axis_index("c")
        s = lax.axis_index("s")

        pltpu.sync_copy(zero_hbm.at[pl.ds(s * RPT, RPT)],
                        agg.at[pl.ds(s * RPT, RPT)])
        plsc.subcore_barrier()

        base = s * RPW

        def _step(g, carry):
            row0 = base + g * CHUNK
            pltpu.sync_copy(src_hbm.at[pl.ds(row0, CHUNK)], sidx)
            pltpu.sync_copy(dst_hbm.at[c, pl.ds(row0, CHUNK)], didx)

            def _fire(j, cr):
                pltpu.async_copy(h_hbm.at[sidx.at[j]], rows.at[j], gsem)
                return cr

            lax.fori_loop(0, CHUNK, _fire, 0)

            def _mid(j, cr):
                pltpu.make_async_copy(h_hbm.at[sidx.at[j]], rows.at[j], gsem).wait()
                pltpu.async_copy(rows.at[j], agg.at[didx.at[j]], ssem, add=True)
                return cr

            lax.fori_loop(0, CHUNK, _mid, 0)

            def _drain(j, cr):
                pltpu.make_async_copy(rows.at[j], agg.at[didx.at[j]], ssem).wait()
                return cr

            lax.fori_loop(0, CHUNK, _drain, 0)
            return carry

        lax.fori_loop(0, STEPS, _step, 0)
        plsc.subcore_barrier()
        pltpu.sync_copy(
            agg.at[pl.ds(s * RPT, RPT)], out_hbm.at[c, pl.ds(s * RPT, RPT)]
        )

    return segsum


_SEG4 = _make_segsum(4)
_SEG16 = _make_segsum(16)
_SEG1 = _make_segsum(1)

_ROW = lambda i: (i, 0)
_FIX = lambda i: (0, 0)


def _tc1_body(x_ref, w_ref, b_ref, h_ref):
    h_ref[...] = jax.nn.relu(
        jnp.dot(x_ref[...], w_ref[...], preferred_element_type=jnp.float32)
        + b_ref[...]
    )


def _tc1(xp, w, b):
    return pl.pallas_call(
        _tc1_body,
        grid=(N // BN,),
        in_specs=[
            pl.BlockSpec((BN, 4), _ROW),
            pl.BlockSpec((4, 4), _FIX),
            pl.BlockSpec((1, 4), _FIX),
        ],
        out_specs=pl.BlockSpec((BN, 4), _ROW),
        out_shape=jax.ShapeDtypeStruct((N, 4), jnp.float32),
    )(xp, w, b)


def _tc2_body(a0, xr, wl, bl, wr, wp, bp, x2_ref, h2_ref):
    x2 = jax.nn.relu(
        jnp.dot(a0[...], wl[...], preferred_element_type=jnp.float32)
        + bl[...]
        + jnp.dot(xr[...], wr[...], preferred_element_type=jnp.float32)
    )
    x2_ref[...] = x2
    h2_ref[...] = jax.nn.relu(
        jnp.dot(x2, wp[...], preferred_element_type=jnp.float32) + bp[...]
    )


def _tc2(a0, xr, wl, bl, wr, wp, bp):
    fa = a0.shape[1]
    return pl.pallas_call(
        _tc2_body,
        grid=(N // BN,),
        in_specs=[
            pl.BlockSpec((BN, fa), _ROW),
            pl.BlockSpec((BN, fa), _ROW),
            pl.BlockSpec((fa, 16), _FIX),
            pl.BlockSpec((1, 16), _FIX),
            pl.BlockSpec((fa, 16), _FIX),
            pl.BlockSpec((16, 16), _FIX),
            pl.BlockSpec((1, 16), _FIX),
        ],
        out_specs=[pl.BlockSpec((BN, 16), _ROW), pl.BlockSpec((BN, 16), _ROW)],
        out_shape=[
            jax.ShapeDtypeStruct((N, 16), jnp.float32),
            jax.ShapeDtypeStruct((N, 16), jnp.float32),
        ],
    )(a0, xr, wl, bl, wr, wp, bp)


def _tc3_body(a0, xr, wl, bl, wr, wp, bp, wo, x3_ref, m_ref):
    x3 = jax.nn.relu(
        jnp.dot(a0[...], wl[...], preferred_element_type=jnp.float32)
        + bl[...]
        + jnp.dot(xr[...], wr[...], preferred_element_type=jnp.float32)
    )
    x3_ref[...] = x3
    t = jax.nn.relu(
        jnp.dot(x3, wp[...], preferred_element_type=jnp.float32) + bp[...]
    )
    m_ref[...] = jnp.dot(t, wo[...], preferred_element_type=jnp.float32)


def _tc3(a0, xr, wl, bl, wr, wp, bp, wo):
    return pl.pallas_call(
        _tc3_body,
        grid=(N // BN,),
        in_specs=[
            pl.BlockSpec((BN, 16), _ROW),
            pl.BlockSpec((BN, 16), _ROW),
            pl.BlockSpec((16, 16), _FIX),
            pl.BlockSpec((1, 16), _FIX),
            pl.BlockSpec((16, 16), _FIX),
            pl.BlockSpec((16, 16), _FIX),
            pl.BlockSpec((1, 16), _FIX),
            pl.BlockSpec((16, 1), _FIX),
        ],
        out_specs=[pl.BlockSpec((BN, 16), _ROW), pl.BlockSpec((BN, 1), _ROW)],
        out_shape=[
            jax.ShapeDtypeStruct((N, 16), jnp.float32),
            jax.ShapeDtypeStruct((N, 1), jnp.float32),
        ],
    )(a0, xr, wl, bl, wr, wp, bp, wo)


def _tc4_body(a0, xr, bl, wr, out_ref):
    out_ref[...] = jax.nn.sigmoid(
        a0[...]
        + bl[...]
        + jnp.dot(xr[...], wr[...], preferred_element_type=jnp.float32)
    )


def _tc4(a0, xr, bl, wr):
    return pl.pallas_call(
        _tc4_body,
        grid=(N // BN,),
        in_specs=[
            pl.BlockSpec((BN, 1), _ROW),
            pl.BlockSpec((BN, 16), _ROW),
            pl.BlockSpec((1, 1), _FIX),
            pl.BlockSpec((16, 1), _FIX),
        ],
        out_specs=pl.BlockSpec((BN, 1), _ROW),
        out_shape=jax.ShapeDtypeStruct((N, 1), jnp.float32),
    )(a0, xr, bl, wr)


def kernel(x, edge_index, Wp1, bp1, Wl1, bl1, Wr1, Wp2, bp2, Wl2, bl2, Wr2,
           Wp3, bp3, Wl3, bl3, Wr3):
    f32 = jnp.float32
    xp = jnp.zeros((N, 4), f32).at[:, :3].set(x)
    wp1 = jnp.zeros((4, 4), f32).at[:3, :3].set(Wp1.T)
    bp1p = jnp.zeros((1, 4), f32).at[0, :3].set(bp1)
    wl1 = jnp.zeros((4, 16), f32).at[:3, :].set(Wl1.T)
    bl1p = bl1.reshape(1, 16)
    wr1 = jnp.zeros((4, 16), f32).at[:3, :].set(Wr1.T)
    wp2, bp2p, wl2, bl2p, wr2 = Wp2.T, bp2.reshape(1, 16), Wl2.T, bl2.reshape(1, 16), Wr2.T
    wp3, bp3p = Wp3.T, bp3.reshape(1, 16)
    wl3, bl3p, wr3 = Wl3.T, bl3.reshape(1, 1), Wr3.T

    src = edge_index[0]
    dst = edge_index[1]
    padn = EP - src.shape[0]
    src2 = jnp.concatenate([src, jnp.zeros((padn,), jnp.int32)]).reshape(EP // CB, CB)
    dstp = jnp.concatenate([dst, jnp.full((padn,), 2 * HNP, jnp.int32)])
    d0 = jnp.where(dstp < HNP, dstp, TRASH)
    d1r = dstp - HNP
    d1 = jnp.where(d1r >= 0, jnp.minimum(d1r, TRASH), TRASH)
    dstm = jnp.stack([d0, d1]).reshape(NC, EP // CB, CB)

    z4 = jnp.zeros((AGR, 4), f32)
    z16 = jnp.zeros((AGR, 16), f32)
    z1 = jnp.zeros((AGR, 1), f32)

    def _merge(agg):
        return jnp.concatenate([agg[0, :HNP], agg[1, : N - HNP]], axis=0)

    h1 = _tc1(xp, wp1, bp1p)
    agg1 = _merge(_SEG4(h1, src2, dstm, z4))
    x2, h2 = _tc2(agg1, xp, wl1, bl1p, wr1, wp2, bp2p)
    agg2 = _merge(_SEG16(h2, src2, dstm, z16))
    x3, m = _tc3(agg2, x2, wl2, bl2p, wr2, wp3, bp3p, wl3)
    agg3 = _merge(_SEG1(m, src2, dstm, z1))
    out = _tc4(agg3, x3, bl3p, wr3)
    return out


# speedup vs baseline: 2.3088x; 2.3088x over previous
import functools

import jax
import jax.numpy as jnp
from jax import lax
from jax.experimental import pallas as pl
from jax.experimental.pallas import tpu as pltpu
from jax.experimental.pallas import tpu_sc as plsc

N = 100000
F = 16
NC, NS, CB = 2, 16, 512
NW = NC * NS
NP = 100224
RPT = NP // NS
TRASH = N
EPW = 102400
EP = EPW * NW
CHUNK = 2
RPW = EPW // CB
STEPS = RPW // CHUNK
BN = 2000


def _make_segsum():
    mesh = plsc.VectorSubcoreMesh(
        core_axis_name="c", subcore_axis_name="s", num_cores=NC, num_subcores=NS
    )

    @functools.partial(
        pl.kernel,
        out_type=jax.ShapeDtypeStruct((NC, NP, F), jnp.float32),
        mesh=mesh,
        scratch_types=[
            pltpu.VMEM((CHUNK, CB), jnp.int32),
            pltpu.VMEM((CHUNK, CB), jnp.int32),
            pltpu.VMEM((CHUNK, CB, F), jnp.float32),
            pltpu.VMEM_SHARED((NP, F), jnp.float32),
            pltpu.SemaphoreType.DMA,
            pltpu.SemaphoreType.DMA,
        ],
        compiler_params=pltpu.CompilerParams(use_tc_tiling_on_sc=False),
    )
    def segsum(h_hbm, src_hbm, dst_hbm, zero_hbm, out_hbm, sidx, didx, rows,
               agg, gsem, ssem):
        c = lax.axis_index("c")
        s = lax.axis_index("s")
        wid = s * NC + c

        pltpu.sync_copy(zero_hbm.at[pl.ds(s * RPT, RPT)],
                        agg.at[pl.ds(s * RPT, RPT)])
        plsc.subcore_barrier()

        base = wid * RPW

        def _step(g, carry):
            row0 = base + g * CHUNK
            pltpu.sync_copy(src_hbm.at[pl.ds(row0, CHUNK)], sidx)
            pltpu.sync_copy(dst_hbm.at[pl.ds(row0, CHUNK)], didx)

            def _fire(j, cr):
                pltpu.async_copy(h_hbm.at[sidx.at[j]], rows.at[j], gsem)
                return cr

            lax.fori_loop(0, CHUNK, _fire, 0)

            def _mid(j, cr):
                pltpu.make_async_copy(h_hbm.at[sidx.at[j]], rows.at[j], gsem).wait()
                pltpu.async_copy(rows.at[j], agg.at[didx.at[j]], ssem, add=True)
                return cr

            lax.fori_loop(0, CHUNK, _mid, 0)

            def _drain(j, cr):
                pltpu.make_async_copy(rows.at[j], agg.at[didx.at[j]], ssem).wait()
                return cr

            lax.fori_loop(0, CHUNK, _drain, 0)
            return carry

        lax.fori_loop(0, STEPS, _step, 0)
        plsc.subcore_barrier()
        pltpu.sync_copy(
            agg.at[pl.ds(s * RPT, RPT)], out_hbm.at[c, pl.ds(s * RPT, RPT)]
        )

    return segsum


_SEGSUM = _make_segsum()

_ROW = lambda i: (i, 0)
_FIX = lambda i: (0, 0)


def _tc1_body(x_ref, w_ref, b_ref, h_ref):
    h_ref[...] = jax.nn.relu(
        jnp.dot(x_ref[...], w_ref[...], preferred_element_type=jnp.float32)
        + b_ref[...]
    )


def _tc1(xp, w, b):
    return pl.pallas_call(
        _tc1_body,
        grid=(N // BN,),
        in_specs=[
            pl.BlockSpec((BN, F), _ROW),
            pl.BlockSpec((F, F), _FIX),
            pl.BlockSpec((1, F), _FIX),
        ],
        out_specs=pl.BlockSpec((BN, F), _ROW),
        out_shape=jax.ShapeDtypeStruct((N, F), jnp.float32),
    )(xp, w, b)


def _tc2_body(a0, a1, xr, wl, bl, wr, wp, bp, x2_ref, h2_ref):
    agg = a0[...] + a1[...]
    x2 = jax.nn.relu(
        jnp.dot(agg, wl[...], preferred_element_type=jnp.float32)
        + bl[...]
        + jnp.dot(xr[...], wr[...], preferred_element_type=jnp.float32)
    )
    x2_ref[...] = x2
    h2_ref[...] = jax.nn.relu(
        jnp.dot(x2, wp[...], preferred_element_type=jnp.float32) + bp[...]
    )


def _tc2(a0, a1, xr, wl, bl, wr, wp, bp):
    return pl.pallas_call(
        _tc2_body,
        grid=(N // BN,),
        in_specs=[
            pl.BlockSpec((BN, F), _ROW),
            pl.BlockSpec((BN, F), _ROW),
            pl.BlockSpec((BN, F), _ROW),
            pl.BlockSpec((F, F), _FIX),
            pl.BlockSpec((1, F), _FIX),
            pl.BlockSpec((F, F), _FIX),
            pl.BlockSpec((F, F), _FIX),
            pl.BlockSpec((1, F), _FIX),
        ],
        out_specs=[pl.BlockSpec((BN, F), _ROW), pl.BlockSpec((BN, F), _ROW)],
        out_shape=[
            jax.ShapeDtypeStruct((N, F), jnp.float32),
            jax.ShapeDtypeStruct((N, F), jnp.float32),
        ],
    )(a0, a1, xr, wl, bl, wr, wp, bp)


def _tc4_body(a0, a1, xr, wl, bl, wr, out_ref):
    agg = a0[...] + a1[...]
    out_ref[...] = jax.nn.sigmoid(
        jnp.dot(agg, wl[...], preferred_element_type=jnp.float32)
        + bl[...]
        + jnp.dot(xr[...], wr[...], preferred_element_type=jnp.float32)
    )


def _tc4(a0, a1, xr, wl, bl, wr):
    return pl.pallas_call(
        _tc4_body,
        grid=(N // BN,),
        in_specs=[
            pl.BlockSpec((BN, F), _ROW),
            pl.BlockSpec((BN, F), _ROW),
            pl.BlockSpec((BN, F), _ROW),
            pl.BlockSpec((F, 1), _FIX),
            pl.BlockSpec((1, 1), _FIX),
            pl.BlockSpec((F, 1), _FIX),
        ],
        out_specs=pl.BlockSpec((BN, 1), _ROW),
        out_shape=jax.ShapeDtypeStruct((N, 1), jnp.float32),
    )(a0, a1, xr, wl, bl, wr)


def kernel(x, edge_index, Wp1, bp1, Wl1, bl1, Wr1, Wp2, bp2, Wl2, bl2, Wr2,
           Wp3, bp3, Wl3, bl3, Wr3):
    f32 = jnp.float32
    xp = jnp.zeros((N, F), f32).at[:, :3].set(x)
    wp1 = jnp.zeros((F, F), f32).at[:3, :3].set(Wp1.T)
    bp1p = jnp.zeros((1, F), f32).at[0, :3].set(bp1)
    wl1 = jnp.zeros((F, F), f32).at[:3, :].set(Wl1.T)
    bl1p = bl1.reshape(1, F)
    wr1 = jnp.zeros((F, F), f32).at[:3, :].set(Wr1.T)
    wp2, bp2p, wl2, bl2p, wr2 = Wp2.T, bp2.reshape(1, F), Wl2.T, bl2.reshape(1, F), Wr2.T
    wp3, bp3p = Wp3.T, bp3.reshape(1, F)
    wl3, bl3p, wr3 = Wl3.T, bl3.reshape(1, 1), Wr3.T

    src = edge_index[0]
    dst = edge_index[1]
    padn = EP - src.shape[0]
    src2 = jnp.concatenate([src, jnp.zeros((padn,), jnp.int32)]).reshape(EP // CB, CB)
    dst2 = jnp.concatenate([dst, jnp.full((padn,), TRASH, jnp.int32)]).reshape(EP // CB, CB)
    zf = jnp.zeros((NP, F), f32)

    h1 = _tc1(xp, wp1, bp1p)
    agg1 = _SEGSUM(h1, src2, dst2, zf)
    x2, h2 = _tc2(agg1[0, :N], agg1[1, :N], xp, wl1, bl1p, wr1, wp2, bp2p)
    agg2 = _SEGSUM(h2, src2, dst2, zf)
    x3, h3 = _tc2(agg2[0, :N], agg2[1, :N], x2, wl2, bl2p, wr2, wp3, bp3p)
    agg3 = _SEGSUM(h3, src2, dst2, zf)
    out = _tc4(agg3[0, :N], agg3[1, :N], x3, wl3, bl3p, wr3)
    return out


# speedup vs baseline: 2.6507x; 1.1481x over previous
import functools

import jax
import jax.numpy as jnp
from jax import lax
from jax.experimental import pallas as pl
from jax.experimental.pallas import tpu as pltpu
from jax.experimental.pallas import tpu_sc as plsc

N = 100000
F = 16
NC, NS, CB = 2, 16, 512
NW = NC * NS
NP = 100224
RPT = NP // NS
TRASH = N
EPW = 102400
EP = EPW * NW
STEPS = EPW // CB
BN = 2000


def _make_segsum():
    mesh = plsc.VectorSubcoreMesh(
        core_axis_name="c", subcore_axis_name="s", num_cores=NC, num_subcores=NS
    )

    @functools.partial(
        pl.kernel,
        out_type=jax.ShapeDtypeStruct((NC, NP, F), jnp.float32),
        mesh=mesh,
        scratch_types=[
            pltpu.VMEM((3, CB), jnp.int32),
            pltpu.VMEM((3, CB), jnp.int32),
            pltpu.VMEM((2, CB, F), jnp.float32),
            pltpu.VMEM_SHARED((NP, F), jnp.float32),
            pltpu.SemaphoreType.DMA,
            pltpu.SemaphoreType.DMA,
            pltpu.SemaphoreType.DMA,
        ],
        compiler_params=pltpu.CompilerParams(use_tc_tiling_on_sc=False),
    )
    def segsum(h_hbm, src_hbm, dst_hbm, zero_hbm, out_hbm, sidx, didx, rows,
               agg, gsem, ssem, isem):
        c = lax.axis_index("c")
        s = lax.axis_index("s")
        wid = s * NC + c

        pltpu.sync_copy(zero_hbm.at[pl.ds(s * RPT, RPT)],
                        agg.at[pl.ds(s * RPT, RPT)])
        plsc.subcore_barrier()

        base = wid * STEPS

        pltpu.async_copy(src_hbm.at[base], sidx.at[0], isem)
        pltpu.async_copy(dst_hbm.at[base], didx.at[0], isem)

        def _iter(g, carry):
            i0 = g % 3
            i1 = (g + 1) % 3
            im = (g - 1) % 3
            p = g % 2
            q = (g + 1) % 2

            @pl.when(g >= 2)
            def _():
                pltpu.make_async_copy(
                    rows.at[p], agg.at[didx.at[i1]], ssem
                ).wait()

            @pl.when(g < STEPS)
            def _():
                pltpu.make_async_copy(src_hbm.at[base + g], sidx.at[i0], isem).wait()
                pltpu.make_async_copy(dst_hbm.at[base + g], didx.at[i0], isem).wait()
                pltpu.async_copy(h_hbm.at[sidx.at[i0]], rows.at[p], gsem)

            @pl.when(g + 1 < STEPS)
            def _():
                pltpu.async_copy(src_hbm.at[base + g + 1], sidx.at[i1], isem)
                pltpu.async_copy(dst_hbm.at[base + g + 1], didx.at[i1], isem)

            @pl.when(jnp.logical_and(g >= 1, g - 1 < STEPS))
            def _():
                pltpu.make_async_copy(
                    h_hbm.at[sidx.at[im]], rows.at[q], gsem
                ).wait()
                pltpu.async_copy(rows.at[q], agg.at[didx.at[im]], ssem, add=True)

            return carry

        lax.fori_loop(0, STEPS + 2, _iter, 0)
        plsc.subcore_barrier()
        pltpu.sync_copy(
            agg.at[pl.ds(s * RPT, RPT)], out_hbm.at[c, pl.ds(s * RPT, RPT)]
        )

    return segsum


_SEGSUM = _make_segsum()

_ROW = lambda i: (i, 0)
_FIX = lambda i: (0, 0)


def _tc1_body(x_ref, w_ref, b_ref, h_ref):
    h_ref[...] = jax.nn.relu(
        jnp.dot(x_ref[...], w_ref[...], preferred_element_type=jnp.float32)
        + b_ref[...]
    )


def _tc1(xp, w, b):
    return pl.pallas_call(
        _tc1_body,
        grid=(N // BN,),
        in_specs=[
            pl.BlockSpec((BN, F), _ROW),
            pl.BlockSpec((F, F), _FIX),
            pl.BlockSpec((1, F), _FIX),
        ],
        out_specs=pl.BlockSpec((BN, F), _ROW),
        out_shape=jax.ShapeDtypeStruct((N, F), jnp.float32),
    )(xp, w, b)


def _tc2_body(a0, a1, xr, wl, bl, wr, wp, bp, x2_ref, h2_ref):
    agg = a0[...] + a1[...]
    x2 = jax.nn.relu(
        jnp.dot(agg, wl[...], preferred_element_type=jnp.float32)
        + bl[...]
        + jnp.dot(xr[...], wr[...], preferred_element_type=jnp.float32)
    )
    x2_ref[...] = x2
    h2_ref[...] = jax.nn.relu(
        jnp.dot(x2, wp[...], preferred_element_type=jnp.float32) + bp[...]
    )


def _tc2(a0, a1, xr, wl, bl, wr, wp, bp):
    return pl.pallas_call(
        _tc2_body,
        grid=(N // BN,),
        in_specs=[
            pl.BlockSpec((BN, F), _ROW),
            pl.BlockSpec((BN, F), _ROW),
            pl.BlockSpec((BN, F), _ROW),
            pl.BlockSpec((F, F), _FIX),
            pl.BlockSpec((1, F), _FIX),
            pl.BlockSpec((F, F), _FIX),
            pl.BlockSpec((F, F), _FIX),
            pl.BlockSpec((1, F), _FIX),
        ],
        out_specs=[pl.BlockSpec((BN, F), _ROW), pl.BlockSpec((BN, F), _ROW)],
        out_shape=[
            jax.ShapeDtypeStruct((N, F), jnp.float32),
            jax.ShapeDtypeStruct((N, F), jnp.float32),
        ],
    )(a0, a1, xr, wl, bl, wr, wp, bp)


def _tc4_body(a0, a1, xr, wl, bl, wr, out_ref):
    agg = a0[...] + a1[...]
    out_ref[...] = jax.nn.sigmoid(
        jnp.dot(agg, wl[...], preferred_element_type=jnp.float32)
        + bl[...]
        + jnp.dot(xr[...], wr[...], preferred_element_type=jnp.float32)
    )


def _tc4(a0, a1, xr, wl, bl, wr):
    return pl.pallas_call(
        _tc4_body,
        grid=(N // BN,),
        in_specs=[
            pl.BlockSpec((BN, F), _ROW),
            pl.BlockSpec((BN, F), _ROW),
            pl.BlockSpec((BN, F), _ROW),
            pl.BlockSpec((F, 1), _FIX),
            pl.BlockSpec((1, 1), _FIX),
            pl.BlockSpec((F, 1), _FIX),
        ],
        out_specs=pl.BlockSpec((BN, 1), _ROW),
        out_shape=jax.ShapeDtypeStruct((N, 1), jnp.float32),
    )(a0, a1, xr, wl, bl, wr)


def kernel(x, edge_index, Wp1, bp1, Wl1, bl1, Wr1, Wp2, bp2, Wl2, bl2, Wr2,
           Wp3, bp3, Wl3, bl3, Wr3):
    f32 = jnp.float32
    xp = jnp.zeros((N, F), f32).at[:, :3].set(x)
    wp1 = jnp.zeros((F, F), f32).at[:3, :3].set(Wp1.T)
    bp1p = jnp.zeros((1, F), f32).at[0, :3].set(bp1)
    wl1 = jnp.zeros((F, F), f32).at[:3, :].set(Wl1.T)
    bl1p = bl1.reshape(1, F)
    wr1 = jnp.zeros((F, F), f32).at[:3, :].set(Wr1.T)
    wp2, bp2p, wl2, bl2p, wr2 = Wp2.T, bp2.reshape(1, F), Wl2.T, bl2.reshape(1, F), Wr2.T
    wp3, bp3p = Wp3.T, bp3.reshape(1, F)
    wl3, bl3p, wr3 = Wl3.T, bl3.reshape(1, 1), Wr3.T

    src = edge_index[0]
    dst = edge_index[1]
    padn = EP - src.shape[0]
    src2 = jnp.concatenate([src, jnp.zeros((padn,), jnp.int32)]).reshape(EP // CB, CB)
    dst2 = jnp.concatenate([dst, jnp.full((padn,), TRASH, jnp.int32)]).reshape(EP // CB, CB)
    zf = jnp.zeros((NP, F), f32)

    h1 = _tc1(xp, wp1, bp1p)
    agg1 = _SEGSUM(h1, src2, dst2, zf)
    x2, h2 = _tc2(agg1[0, :N], agg1[1, :N], xp, wl1, bl1p, wr1, wp2, bp2p)
    agg2 = _SEGSUM(h2, src2, dst2, zf)
    x3, h3 = _tc2(agg2[0, :N], agg2[1, :N], x2, wl2, bl2p, wr2, wp3, bp3p)
    agg3 = _SEGSUM(h3, src2, dst2, zf)
    out = _tc4(agg3[0, :N], agg3[1, :N], x3, wl3, bl3p, wr3)
    return out


# speedup vs baseline: 2.8425x; 1.0724x over previous
import functools

import jax
import jax.numpy as jnp
from jax import lax
from jax.experimental import pallas as pl
from jax.experimental.pallas import tpu as pltpu
from jax.experimental.pallas import tpu_sc as plsc

N = 100000
F = 16
NC, NS, CB = 2, 16, 512
NW = NC * NS
NP = 100224
RPT = NP // NS
TRASH = N
EP = 3276800
SPF = 294
SPS = 106
BN = 2000


def _make_segsum():
    mesh = plsc.VectorSubcoreMesh(
        core_axis_name="c", subcore_axis_name="s", num_cores=NC, num_subcores=NS
    )

    @functools.partial(
        pl.kernel,
        out_type=jax.ShapeDtypeStruct((NC, NP, F), jnp.float32),
        mesh=mesh,
        scratch_types=[
            pltpu.VMEM((3, CB), jnp.int32),
            pltpu.VMEM((3, CB), jnp.int32),
            pltpu.VMEM((2, CB, F), jnp.float32),
            pltpu.VMEM_SHARED((NP, F), jnp.float32),
            pltpu.SemaphoreType.DMA,
            pltpu.SemaphoreType.DMA,
            pltpu.SemaphoreType.DMA,
        ],
        compiler_params=pltpu.CompilerParams(use_tc_tiling_on_sc=False),
    )
    def segsum(h_hbm, src_hbm, dst_hbm, zero_hbm, out_hbm, sidx, didx, rows,
               agg, gsem, ssem, isem):
        c = lax.axis_index("c")
        s = lax.axis_index("s")

        pltpu.sync_copy(zero_hbm.at[pl.ds(s * RPT, RPT)],
                        agg.at[pl.ds(s * RPT, RPT)])
        plsc.subcore_barrier()

        steps = jnp.where(c == 0, SPF, SPS)
        base = jnp.where(c == 0, s * SPF, NS * SPF + s * SPS)

        pltpu.async_copy(src_hbm.at[base], sidx.at[0], isem)
        pltpu.async_copy(dst_hbm.at[base], didx.at[0], isem)

        def _iter(g, carry):
            i0 = g % 3
            i1 = (g + 1) % 3
            im = (g - 1) % 3
            p = g % 2
            q = (g + 1) % 2

            @pl.when(g >= 2)
            def _():
                pltpu.make_async_copy(
                    rows.at[p], agg.at[didx.at[i1]], ssem
                ).wait()

            @pl.when(g < steps)
            def _():
                pltpu.make_async_copy(src_hbm.at[base + g], sidx.at[i0], isem).wait()
                pltpu.make_async_copy(dst_hbm.at[base + g], didx.at[i0], isem).wait()
                pltpu.async_copy(h_hbm.at[sidx.at[i0]], rows.at[p], gsem)

            @pl.when(g + 1 < steps)
            def _():
                pltpu.async_copy(src_hbm.at[base + g + 1], sidx.at[i1], isem)
                pltpu.async_copy(dst_hbm.at[base + g + 1], didx.at[i1], isem)

            @pl.when(jnp.logical_and(g >= 1, g - 1 < steps))
            def _():
                pltpu.make_async_copy(
                    h_hbm.at[sidx.at[im]], rows.at[q], gsem
                ).wait()
                pltpu.async_copy(rows.at[q], agg.at[didx.at[im]], ssem, add=True)

            return carry

        lax.fori_loop(0, steps + 2, _iter, 0)
        plsc.subcore_barrier()
        pltpu.sync_copy(
            agg.at[pl.ds(s * RPT, RPT)], out_hbm.at[c, pl.ds(s * RPT, RPT)]
        )

    return segsum


_SEGSUM = _make_segsum()

_ROW = lambda i: (i, 0)
_FIX = lambda i: (0, 0)


def _tc1_body(x_ref, w_ref, b_ref, h_ref):
    h_ref[...] = jax.nn.relu(
        jnp.dot(x_ref[...], w_ref[...], preferred_element_type=jnp.float32)
        + b_ref[...]
    )


def _tc1(xp, w, b):
    return pl.pallas_call(
        _tc1_body,
        grid=(N // BN,),
        in_specs=[
            pl.BlockSpec((BN, F), _ROW),
            pl.BlockSpec((F, F), _FIX),
            pl.BlockSpec((1, F), _FIX),
        ],
        out_specs=pl.BlockSpec((BN, F), _ROW),
        out_shape=jax.ShapeDtypeStruct((N, F), jnp.float32),
    )(xp, w, b)


def _tc2_body(a0, a1, xr, wl, bl, wr, wp, bp, x2_ref, h2_ref):
    agg = a0[...] + a1[...]
    x2 = jax.nn.relu(
        jnp.dot(agg, wl[...], preferred_element_type=jnp.float32)
        + bl[...]
        + jnp.dot(xr[...], wr[...], preferred_element_type=jnp.float32)
    )
    x2_ref[...] = x2
    h2_ref[...] = jax.nn.relu(
        jnp.dot(x2, wp[...], preferred_element_type=jnp.float32) + bp[...]
    )


def _tc2(a0, a1, xr, wl, bl, wr, wp, bp):
    return pl.pallas_call(
        _tc2_body,
        grid=(N // BN,),
        in_specs=[
            pl.BlockSpec((BN, F), _ROW),
            pl.BlockSpec((BN, F), _ROW),
            pl.BlockSpec((BN, F), _ROW),
            pl.BlockSpec((F, F), _FIX),
            pl.BlockSpec((1, F), _FIX),
            pl.BlockSpec((F, F), _FIX),
            pl.BlockSpec((F, F), _FIX),
            pl.BlockSpec((1, F), _FIX),
        ],
        out_specs=[pl.BlockSpec((BN, F), _ROW), pl.BlockSpec((BN, F), _ROW)],
        out_shape=[
            jax.ShapeDtypeStruct((N, F), jnp.float32),
            jax.ShapeDtypeStruct((N, F), jnp.float32),
        ],
    )(a0, a1, xr, wl, bl, wr, wp, bp)


def _tc4_body(a0, a1, xr, wl, bl, wr, out_ref):
    agg = a0[...] + a1[...]
    out_ref[...] = jax.nn.sigmoid(
        jnp.dot(agg, wl[...], preferred_element_type=jnp.float32)
        + bl[...]
        + jnp.dot(xr[...], wr[...], preferred_element_type=jnp.float32)
    )


def _tc4(a0, a1, xr, wl, bl, wr):
    return pl.pallas_call(
        _tc4_body,
        grid=(N // BN,),
        in_specs=[
            pl.BlockSpec((BN, F), _ROW),
            pl.BlockSpec((BN, F), _ROW),
            pl.BlockSpec((BN, F), _ROW),
            pl.BlockSpec((F, 1), _FIX),
            pl.BlockSpec((1, 1), _FIX),
            pl.BlockSpec((F, 1), _FIX),
        ],
        out_specs=pl.BlockSpec((BN, 1), _ROW),
        out_shape=jax.ShapeDtypeStruct((N, 1), jnp.float32),
    )(a0, a1, xr, wl, bl, wr)


def kernel(x, edge_index, Wp1, bp1, Wl1, bl1, Wr1, Wp2, bp2, Wl2, bl2, Wr2,
           Wp3, bp3, Wl3, bl3, Wr3):
    f32 = jnp.float32
    xp = jnp.zeros((N, F), f32).at[:, :3].set(x)
    wp1 = jnp.zeros((F, F), f32).at[:3, :3].set(Wp1.T)
    bp1p = jnp.zeros((1, F), f32).at[0, :3].set(bp1)
    wl1 = jnp.zeros((F, F), f32).at[:3, :].set(Wl1.T)
    bl1p = bl1.reshape(1, F)
    wr1 = jnp.zeros((F, F), f32).at[:3, :].set(Wr1.T)
    wp2, bp2p, wl2, bl2p, wr2 = Wp2.T, bp2.reshape(1, F), Wl2.T, bl2.reshape(1, F), Wr2.T
    wp3, bp3p = Wp3.T, bp3.reshape(1, F)
    wl3, bl3p, wr3 = Wl3.T, bl3.reshape(1, 1), Wr3.T

    src = edge_index[0]
    dst = edge_index[1]
    padn = EP - src.shape[0]
    src2 = jnp.concatenate([src, jnp.zeros((padn,), jnp.int32)]).reshape(EP // CB, CB)
    dst2 = jnp.concatenate([dst, jnp.full((padn,), TRASH, jnp.int32)]).reshape(EP // CB, CB)
    zf = jnp.zeros((NP, F), f32)

    h1 = _tc1(xp, wp1, bp1p)
    agg1 = _SEGSUM(h1, src2, dst2, zf)
    x2, h2 = _tc2(agg1[0, :N], agg1[1, :N], xp, wl1, bl1p, wr1, wp2, bp2p)
    agg2 = _SEGSUM(h2, src2, dst2, zf)
    x3, h3 = _tc2(agg2[0, :N], agg2[1, :N], x2, wl2, bl2p, wr2, wp3, bp3p)
    agg3 = _SEGSUM(h3, src2, dst2, zf)
    out = _tc4(agg3[0, :N], agg3[1, :N], x3, wl3, bl3p, wr3)
    return out
